# Initial kernel scaffold; baseline (speedup 1.0000x reference)
import jax
import jax.numpy as jnp
from jax.experimental import pallas as pl


def kernel(input_ids, table):
    raise NotImplementedError("write your pallas kernel here")


# speedup vs baseline: 1.5717x; 1.5717x over previous
import functools

import jax
import jax.numpy as jnp
from jax import lax
from jax.experimental import pallas as pl
from jax.experimental.pallas import tpu as pltpu
from jax.experimental.pallas import tpu_sc as plsc

NUM_EMB = 1_000_000
DIM = 64
BATCH = 16384
HIST = 50
TOTAL = BATCH * HIST

NUM_CORES = 2
NUM_SUBCORES = 16
NW = NUM_CORES * NUM_SUBCORES
PER_W = TOTAL // NW
CHUNK = 128
NCHUNK = PER_W // CHUNK

_mesh = plsc.VectorSubcoreMesh(core_axis_name="c", subcore_axis_name="s")


@functools.partial(
    pl.kernel,
    mesh=_mesh,
    out_type=jax.ShapeDtypeStruct((TOTAL, DIM), jnp.float32),
    scratch_types=[
        pltpu.VMEM((CHUNK,), jnp.int32),
        pltpu.VMEM((CHUNK, DIM), jnp.float32),
        pltpu.SemaphoreType.DMA,
    ],
    compiler_params=pltpu.CompilerParams(use_tc_tiling_on_sc=False),
)
def _gather_rows(ids_hbm, table_hbm, out_hbm, idx_v, rows_v, sem):
    wid = lax.axis_index("s") * NUM_CORES + lax.axis_index("c")
    w_base = wid * PER_W

    def body(g, carry):
        base = w_base + g * CHUNK
        pltpu.sync_copy(ids_hbm.at[pl.ds(base, CHUNK)], idx_v)
        pltpu.async_copy(table_hbm.at[idx_v], rows_v, sem).wait()
        pltpu.sync_copy(rows_v, out_hbm.at[pl.ds(base, CHUNK)])
        return carry

    lax.fori_loop(0, NCHUNK, body, 0)


def kernel(input_ids, table):
    ids = input_ids.reshape(TOTAL).astype(jnp.int32)
    out = _gather_rows(ids, table)
    return out.reshape(BATCH, HIST, DIM)


# speedup vs baseline: 1.8771x; 1.1943x over previous
import functools

import jax
import jax.numpy as jnp
from jax import lax
from jax.experimental import pallas as pl
from jax.experimental.pallas import tpu as pltpu
from jax.experimental.pallas import tpu_sc as plsc

NUM_EMB = 1_000_000
DIM = 64
BATCH = 16384
HIST = 50
TOTAL = BATCH * HIST

NUM_CORES = 2
NUM_SUBCORES = 16
NW = NUM_CORES * NUM_SUBCORES
PER_W = TOTAL // NW
CHUNK = 128
NCHUNK = PER_W // CHUNK

NBUF = 4
CH = 256
KG = CH // CHUNK
NSLAB = PER_W // CH
T_OUTER = NSLAB // NBUF

_mesh = plsc.VectorSubcoreMesh(core_axis_name="c", subcore_axis_name="s")


@functools.partial(
    pl.kernel,
    mesh=_mesh,
    out_type=jax.ShapeDtypeStruct((TOTAL, DIM), jnp.float32),
    scratch_types=[
        pltpu.VMEM((NCHUNK, CHUNK), jnp.int32),
        pltpu.VMEM((NBUF, CH, DIM), jnp.float32),
        pltpu.SemaphoreType.DMA((NBUF,)),
        pltpu.SemaphoreType.DMA((NBUF,)),
    ],
    compiler_params=pltpu.CompilerParams(use_tc_tiling_on_sc=False),
)
def _gather_rows(ids_hbm, table_hbm, out_hbm, idx_v, rows_v, gsem, ssem):
    wid = lax.axis_index("s") * NUM_CORES + lax.axis_index("c")
    w_base = wid * PER_W

    pltpu.sync_copy(ids_hbm.at[wid], idx_v)

    def fire_gathers(s, b):
        for k in range(KG):
            pltpu.async_copy(
                table_hbm.at[idx_v.at[s * KG + k]],
                rows_v.at[b].at[pl.ds(k * CHUNK, CHUNK)],
                gsem.at[b])

    def drain_gathers(s, b):
        for k in range(KG):
            pltpu.make_async_copy(
                table_hbm.at[idx_v.at[s * KG + k]],
                rows_v.at[b].at[pl.ds(k * CHUNK, CHUNK)],
                gsem.at[b]).wait()

    def fire_store(s, b):
        pltpu.async_copy(
            rows_v.at[b], out_hbm.at[pl.ds(w_base + s * CH, CH)], ssem.at[b])

    def wait_store(s, b):
        pltpu.make_async_copy(
            rows_v.at[b], out_hbm.at[pl.ds(w_base + s * CH, CH)], ssem.at[b]).wait()

    fire_gathers(0, 0)
    fire_gathers(1, 1)

    drain_gathers(0, 0)
    fire_store(0, 0)
    fire_gathers(2, 2)

    drain_gathers(1, 1)
    fire_store(1, 1)
    fire_gathers(3, 3)

    def outer(t, carry):
        for j in range(NBUF):
            s = t * NBUF + j + 2
            b = (j + 2) % NBUF
            drain_gathers(s, b)
            fire_store(s, b)
            wait_store(s - 2, j)
            fire_gathers(s + 2, j)
        return carry

    lax.fori_loop(0, (NSLAB - NBUF) // NBUF, outer, 0)

    drain_gathers(NSLAB - 2, (NSLAB - 2) % NBUF)
    fire_store(NSLAB - 2, (NSLAB - 2) % NBUF)
    drain_gathers(NSLAB - 1, (NSLAB - 1) % NBUF)
    fire_store(NSLAB - 1, (NSLAB - 1) % NBUF)

    for j in range(NBUF):
        wait_store(NSLAB - NBUF + j, j)


def kernel(input_ids, table):
    ids = input_ids.reshape(NW, NCHUNK, CHUNK).astype(jnp.int32)
    out = _gather_rows(ids, table)
    return out.reshape(BATCH, HIST, DIM)
